# Initial kernel scaffold; baseline (speedup 1.0000x reference)
import jax
import jax.numpy as jnp
from jax.experimental import pallas as pl


def kernel(x, y, W1, b1, gamma, beta, W2, b2):
    raise NotImplementedError("write your pallas kernel here")


# speedup vs baseline: 1.0467x; 1.0467x over previous
import functools
import jax
import jax.numpy as jnp
from jax.experimental import pallas as pl
from jax.experimental.pallas import tpu as pltpu

_X_SIZE = 512.0
_Y_SIZE = 512.0
_TB = 2048


def _dense_body(x_ref, y_ref, W1_ref, b1_ref, gamma_ref, beta_ref, W2_ref, b2_ref, out_ref):
    ax = (x_ref[...].astype(jnp.float32) - _X_SIZE * 0.5) * (1.0 / _X_SIZE)
    ay = (y_ref[...].astype(jnp.float32) - _Y_SIZE * 0.5) * (1.0 / _Y_SIZE)
    u = W1_ref[0:1, :]
    v = W1_ref[1:2, :]
    h = ax * u + ay * v + b1_ref[...]
    mu = jnp.mean(h, axis=-1, keepdims=True)
    d = h - mu
    var = jnp.mean(d * d, axis=-1, keepdims=True)
    hn = d * jax.lax.rsqrt(var + 1e-5) * gamma_ref[...] + beta_ref[...]
    hr = jnp.maximum(hn, 0.0)
    out_ref[...] = (
        jnp.dot(hr, W2_ref[...], preferred_element_type=jnp.float32) + b2_ref[...]
    )


def kernel(x, y, W1, b1, gamma, beta, W2, b2):
    B, L = x.shape
    N = B * L
    E = W2.shape[1]
    D = W2.shape[0]
    xc = x.reshape(N, 1)
    yc = y.reshape(N, 1)
    grid = (N // _TB,)
    out = pl.pallas_call(
        _dense_body,
        grid=grid,
        in_specs=[
            pl.BlockSpec((_TB, 1), lambda i: (i, 0)),
            pl.BlockSpec((_TB, 1), lambda i: (i, 0)),
            pl.BlockSpec((2, D), lambda i: (0, 0)),
            pl.BlockSpec((1, D), lambda i: (0, 0)),
            pl.BlockSpec((1, D), lambda i: (0, 0)),
            pl.BlockSpec((1, D), lambda i: (0, 0)),
            pl.BlockSpec((D, E), lambda i: (0, 0)),
            pl.BlockSpec((1, E), lambda i: (0, 0)),
        ],
        out_specs=pl.BlockSpec((_TB, E), lambda i: (i, 0)),
        out_shape=jax.ShapeDtypeStruct((N, E), jnp.float32),
    )(xc, yc, W1, b1.reshape(1, D), gamma.reshape(1, D), beta.reshape(1, D), W2, b2.reshape(1, E))
    return out.reshape(B, L, E)


# speedup vs baseline: 1.0504x; 1.0035x over previous
import functools
import jax
import jax.numpy as jnp
from jax.experimental import pallas as pl
from jax.experimental.pallas import tpu as pltpu

_X_SIZE = 512.0
_Y_SIZE = 512.0
_TB = 2048


def _dense_body(x_ref, y_ref, W1_ref, b1_ref, gamma_ref, beta_ref, W2_ref, b2_ref, out_ref):
    ax = (x_ref[...].astype(jnp.float32) - _X_SIZE * 0.5) * (1.0 / _X_SIZE)
    ay = (y_ref[...].astype(jnp.float32) - _Y_SIZE * 0.5) * (1.0 / _Y_SIZE)
    u = W1_ref[0:1, :]
    v = W1_ref[1:2, :]
    h = ax * u + ay * v + b1_ref[...]
    mu = jnp.mean(h, axis=-1, keepdims=True)
    d = h - mu
    var = jnp.mean(d * d, axis=-1, keepdims=True)
    hn = d * jax.lax.rsqrt(var + 1e-5) * gamma_ref[...] + beta_ref[...]
    hr = jnp.maximum(hn, 0.0)
    out_ref[...] = (
        jnp.dot(
            hr.astype(jnp.bfloat16),
            W2_ref[...].astype(jnp.bfloat16),
            preferred_element_type=jnp.float32,
        )
        + b2_ref[...]
    )


def kernel(x, y, W1, b1, gamma, beta, W2, b2):
    B, L = x.shape
    N = B * L
    E = W2.shape[1]
    D = W2.shape[0]
    xc = x.reshape(N, 1)
    yc = y.reshape(N, 1)
    grid = (N // _TB,)
    out = pl.pallas_call(
        _dense_body,
        grid=grid,
        in_specs=[
            pl.BlockSpec((_TB, 1), lambda i: (i, 0)),
            pl.BlockSpec((_TB, 1), lambda i: (i, 0)),
            pl.BlockSpec((2, D), lambda i: (0, 0)),
            pl.BlockSpec((1, D), lambda i: (0, 0)),
            pl.BlockSpec((1, D), lambda i: (0, 0)),
            pl.BlockSpec((1, D), lambda i: (0, 0)),
            pl.BlockSpec((D, E), lambda i: (0, 0)),
            pl.BlockSpec((1, E), lambda i: (0, 0)),
        ],
        out_specs=pl.BlockSpec((_TB, E), lambda i: (i, 0)),
        out_shape=jax.ShapeDtypeStruct((N, E), jnp.float32),
    )(xc, yc, W1, b1.reshape(1, D), gamma.reshape(1, D), beta.reshape(1, D), W2, b2.reshape(1, E))
    return out.reshape(B, L, E)
